# baseline (speedup 1.0000x reference)
import jax
import jax.numpy as jnp
from jax import lax
from jax.experimental import pallas as pl

_SIGMA = 0.5
_IOU_THRESH = 0.7
_NPAD = 5120
_BR = 256
_NC = 512
_PAD_KEY = jnp.iinfo(jnp.int32).min


def _iou_chunk(cols_ref, rx1, ry1, rx2, ry2, r_area, csl):
    cx1 = cols_ref[0:1, csl]
    cy1 = cols_ref[1:2, csl]
    cx2 = cols_ref[2:3, csl]
    cy2 = cols_ref[3:4, csl]
    c_area = (cx2 - cx1) * (cy2 - cy1)
    xx1 = jnp.maximum(rx1, cx1)
    yy1 = jnp.maximum(ry1, cy1)
    xx2 = jnp.minimum(rx2, cx2)
    yy2 = jnp.minimum(ry2, cy2)
    w = jnp.maximum(xx2 - xx1, 0.0)
    h = jnp.maximum(yy2 - yy1, 0.0)
    inter = w * h
    union = r_area + c_area - inter
    return inter / union


def _nms_body(cols_ref, rows_ref, keys_ref, out_ref):
    i = pl.program_id(0)
    cm = i // 2
    rx1 = rows_ref[:, 0:1]
    ry1 = rows_ref[:, 1:2]
    rx2 = rows_ref[:, 2:3]
    ry2 = rows_ref[:, 3:4]
    rs = rows_ref[:, 4:5]
    r_area = (rx2 - rx1) * (ry2 - ry1)
    u2r = lax.bitcast_convert_type(rows_ref[:, 5:6], jnp.int32)
    ridx = lax.broadcasted_iota(jnp.int32, (_BR, 1), 0) + i * _BR

    out_ref[...] = jnp.zeros((_BR, 1), jnp.float32)
    acc = jnp.zeros((_BR, 1), jnp.float32)
    for c in range(_NPAD // _NC):
        sl = slice(c * _NC, (c + 1) * _NC)
        iou = _iou_chunk(cols_ref, rx1, ry1, rx2, ry2, r_area, sl)
        u2c = jnp.where(c < cm, keys_ref[1:2, sl], keys_ref[0:1, sl])
        masked = jnp.where(u2c > u2r, iou, 0.0)
        acc = jnp.maximum(acc, jnp.max(masked, axis=1, keepdims=True))

        def tie_body(c=c, sl=sl, iou=iou):
            u2 = keys_ref[0:1, sl]
            cidx = lax.broadcasted_iota(jnp.int32, (1, _NC), 1) + c * _NC
            tie = (u2 == u2r) & (cidx < ridx)
            tm = jnp.max(jnp.where(tie, iou, 0.0), axis=1, keepdims=True)
            out_ref[...] = tm

        pl.when(c == cm)(tie_body)

    acc = jnp.maximum(acc, out_ref[...])
    decay = jnp.exp(-(acc * acc) / _SIGMA)
    keep = (acc <= _IOU_THRESH).astype(jnp.float32)
    out_ref[...] = rs * decay * keep


@jax.jit
def _nms_scores_pallas(boxes, scores):
    n = scores.shape[0]
    pad = _NPAD - n
    b = jnp.pad(boxes, ((0, pad), (0, 0)))
    u2 = lax.bitcast_convert_type(scores, jnp.int32) * 2
    u2p = jnp.pad(u2, (0, pad), constant_values=_PAD_KEY)
    s = jnp.pad(scores, (0, pad))
    cols = jnp.zeros((8, _NPAD), jnp.float32)
    cols = cols.at[0:4, :].set(b.T)
    keys = jnp.zeros((8, _NPAD), jnp.int32)
    keys = keys.at[0, :].set(u2p).at[1, :].set(u2p + 1)
    rows = jnp.zeros((_NPAD, 8), jnp.float32)
    rows = rows.at[:, 0:4].set(b).at[:, 4].set(s)
    rows = rows.at[:, 5].set(lax.bitcast_convert_type(u2p, jnp.float32))

    out = pl.pallas_call(
        _nms_body,
        grid=(_NPAD // _BR,),
        in_specs=[
            pl.BlockSpec((8, _NPAD), lambda i: (0, 0)),
            pl.BlockSpec((_BR, 8), lambda i: (i, 0)),
            pl.BlockSpec((8, _NPAD), lambda i: (0, 0)),
        ],
        out_specs=pl.BlockSpec((_BR, 1), lambda i: (i, 0)),
        out_shape=jax.ShapeDtypeStruct((_NPAD, 1), jnp.float32),
    )(cols, rows, keys)
    return out[:n, 0]


def kernel(boxes, scores, k):
    new_scores = _nms_scores_pallas(boxes, scores)
    topk_vals, topk_idx = jax.lax.top_k(new_scores, 150)
    return new_scores, topk_vals, topk_idx


# speedup vs baseline: 1.1657x; 1.1657x over previous
import jax
import jax.numpy as jnp
from jax import lax
from jax.experimental import pallas as pl

_SIGMA = 0.5
_IOU_THRESH = 0.7
_NPAD = 5120
_BR = 512
_NC = 512


def _nms_body(cols_ref, rows_ref, out_ref):
    i = pl.program_id(0)
    rx1 = rows_ref[:, 0:1]
    ry1 = rows_ref[:, 1:2]
    rx2 = rows_ref[:, 2:3]
    ry2 = rows_ref[:, 3:4]
    rs = rows_ref[:, 4:5]
    r_area = (rx2 - rx1) * (ry2 - ry1)
    ridx = lax.broadcasted_iota(jnp.int32, (_BR, 1), 0) + i * _BR

    acc = jnp.zeros((_BR, 1), jnp.float32)
    for c in range(_NPAD // _NC):
        sl = slice(c * _NC, (c + 1) * _NC)
        cx1 = cols_ref[0:1, sl]
        cy1 = cols_ref[1:2, sl]
        cx2 = cols_ref[2:3, sl]
        cy2 = cols_ref[3:4, sl]
        cs = cols_ref[4:5, sl]
        c_area = (cx2 - cx1) * (cy2 - cy1)
        xx1 = jnp.maximum(rx1, cx1)
        yy1 = jnp.maximum(ry1, cy1)
        xx2 = jnp.minimum(rx2, cx2)
        yy2 = jnp.minimum(ry2, cy2)
        w = jnp.maximum(xx2 - xx1, 0.0)
        h = jnp.maximum(yy2 - yy1, 0.0)
        inter = w * h
        union = r_area + c_area - inter
        iou = inter / jnp.maximum(union, 1e-8)
        cidx = lax.broadcasted_iota(jnp.int32, (1, _NC), 1) + c * _NC
        mask = (cs > rs) | ((cs == rs) & (cidx < ridx))
        acc = jnp.maximum(
            acc, jnp.max(jnp.where(mask, iou, 0.0), axis=1, keepdims=True))

    decay = jnp.exp(-(acc * acc) / _SIGMA)
    keep = (acc <= _IOU_THRESH).astype(jnp.float32)
    out_ref[...] = rs * decay * keep


@jax.jit
def _nms_scores_pallas(boxes, scores):
    n = scores.shape[0]
    pad = _NPAD - n
    b = jnp.pad(boxes, ((0, pad), (0, 0)))
    s = jnp.pad(scores, (0, pad), constant_values=-1.0)
    cols = jnp.zeros((8, _NPAD), jnp.float32)
    cols = cols.at[0:4, :].set(b.T).at[4, :].set(s)
    rows = jnp.zeros((_NPAD, 8), jnp.float32)
    rows = rows.at[:, 0:4].set(b).at[:, 4].set(s)

    out = pl.pallas_call(
        _nms_body,
        grid=(_NPAD // _BR,),
        in_specs=[
            pl.BlockSpec((8, _NPAD), lambda i: (0, 0)),
            pl.BlockSpec((_BR, 8), lambda i: (i, 0)),
        ],
        out_specs=pl.BlockSpec((_BR, 1), lambda i: (i, 0)),
        out_shape=jax.ShapeDtypeStruct((_NPAD, 1), jnp.float32),
    )(cols, rows)
    return out[:n, 0]


def kernel(boxes, scores, k):
    new_scores = _nms_scores_pallas(boxes, scores)
    topk_vals, topk_idx = jax.lax.top_k(new_scores, 150)
    return new_scores, topk_vals, topk_idx
